# Initial kernel scaffold; baseline (speedup 1.0000x reference)
import jax
import jax.numpy as jnp
from jax.experimental import pallas as pl


def kernel(x, edge_index, kernel_idx, W1, gamma1, beta1, W2, gamma2, beta2):
    raise NotImplementedError("write your pallas kernel here")


# speedup vs baseline: 2.7992x; 2.7992x over previous
import functools

import jax
import jax.numpy as jnp
from jax import lax
from jax.experimental import pallas as pl
from jax.experimental.pallas import tpu as pltpu
from jax.experimental.pallas import tpu_sc as plsc

_EPS = 1e-5


def _xw_body(x_ref, w_ref, o_ref):
    o_ref[0] = jnp.dot(x_ref[...], w_ref[0], preferred_element_type=jnp.float32)


def _einsum_xw(x, w, bn):
    n, c = x.shape
    k = w.shape[0]
    nb = n // bn
    return pl.pallas_call(
        _xw_body,
        grid=(nb, k),
        in_specs=[
            pl.BlockSpec((bn, c), lambda i, j: (i, 0)),
            pl.BlockSpec((1, c, c), lambda i, j: (j, 0, 0)),
        ],
        out_specs=pl.BlockSpec((1, bn, c), lambda i, j: (j, i, 0)),
        out_shape=jax.ShapeDtypeStruct((k, n, c), jnp.float32),
    )(x, w)


def _bnxw_body(h_ref, coef_ref, w_ref, o_ref):
    c1 = coef_ref[0:1, :]
    c2 = coef_ref[1:2, :]
    a = jnp.maximum(h_ref[...] * c1 + c2, 0.0)
    o_ref[0] = jnp.dot(a, w_ref[0], preferred_element_type=jnp.float32)


def _einsum_bn_xw(h, coef, w, bn):
    n, c = h.shape
    k = w.shape[0]
    nb = n // bn
    return pl.pallas_call(
        _bnxw_body,
        grid=(nb, k),
        in_specs=[
            pl.BlockSpec((bn, c), lambda i, j: (i, 0)),
            pl.BlockSpec((8, c), lambda i, j: (0, 0)),
            pl.BlockSpec((1, c, c), lambda i, j: (j, 0, 0)),
        ],
        out_specs=pl.BlockSpec((1, bn, c), lambda i, j: (j, i, 0)),
        out_shape=jax.ShapeDtypeStruct((k, n, c), jnp.float32),
    )(h, coef, w)


def _stats_body(n_nodes, p_ref, g_ref, b_ref, h_ref, coef_ref):
    h = p_ref[0] + p_ref[1]
    h_ref[...] = h

    @pl.when(pl.program_id(0) == 0)
    def _init():
        coef_ref[...] = jnp.zeros_like(coef_ref)

    coef_ref[0:1, :] = coef_ref[0:1, :] + jnp.sum(h, axis=0, keepdims=True)
    coef_ref[1:2, :] = coef_ref[1:2, :] + jnp.sum(h * h, axis=0, keepdims=True)

    @pl.when(pl.program_id(0) == pl.num_programs(0) - 1)
    def _finalize():
        tot = coef_ref[0:1, :]
        totsq = coef_ref[1:2, :]
        mean = tot / n_nodes
        var = totsq / n_nodes - mean * mean
        c1 = g_ref[...] * lax.rsqrt(var + _EPS)
        c2 = b_ref[...] - mean * c1
        coef_ref[0:1, :] = c1
        coef_ref[1:2, :] = c2


def _stats(p, gamma, beta, bn):
    _, n, c = p.shape
    nb = n // bn
    return pl.pallas_call(
        functools.partial(_stats_body, float(n)),
        grid=(nb,),
        in_specs=[
            pl.BlockSpec((2, bn, c), lambda i: (0, i, 0)),
            pl.BlockSpec((1, c), lambda i: (0, 0)),
            pl.BlockSpec((1, c), lambda i: (0, 0)),
        ],
        out_specs=[
            pl.BlockSpec((bn, c), lambda i: (i, 0)),
            pl.BlockSpec((8, c), lambda i: (0, 0)),
        ],
        out_shape=[
            jax.ShapeDtypeStruct((n, c), jnp.float32),
            jax.ShapeDtypeStruct((8, c), jnp.float32),
        ],
    )(p, gamma.reshape(1, c), beta.reshape(1, c))


def _final_body(h_ref, x_ref, coef_ref, o_ref):
    c1 = coef_ref[0:1, :]
    c2 = coef_ref[1:2, :]
    o_ref[...] = jnp.maximum(h_ref[...] * c1 + c2 + x_ref[...], 0.0)


def _final(h, x, coef, bn):
    n, c = h.shape
    nb = n // bn
    return pl.pallas_call(
        _final_body,
        grid=(nb,),
        in_specs=[
            pl.BlockSpec((bn, c), lambda i: (i, 0)),
            pl.BlockSpec((bn, c), lambda i: (i, 0)),
            pl.BlockSpec((8, c), lambda i: (0, 0)),
        ],
        out_specs=pl.BlockSpec((bn, c), lambda i: (i, 0)),
        out_shape=jax.ShapeDtypeStruct((n, c), jnp.float32),
    )(h, x, coef)


def _make_sc_gather_scatter(kn, n_nodes, c, n_edges):
    GRP = 2
    SUP = 128 * GRP
    nsuper = n_edges // SUP
    info = plsc.get_sparse_core_info()
    ncores, nsub = info.num_cores, info.num_subcores
    nw = ncores * nsub
    iters = (nsuper + nw - 1) // nw
    rpt = (n_nodes // nsub) // 8 * 8
    tail = n_nodes - nsub * rpt

    mesh = plsc.VectorSubcoreMesh(core_axis_name="c", subcore_axis_name="s")

    @functools.partial(
        pl.kernel,
        out_type=jax.ShapeDtypeStruct((ncores, n_nodes, c), jnp.float32),
        mesh=mesh,
        scratch_types=[
            pltpu.VMEM((SUP,), jnp.int32),
            pltpu.VMEM((SUP,), jnp.int32),
            [pltpu.VMEM((128,), jnp.int32) for _ in range(GRP)],
            [pltpu.VMEM((128,), jnp.int32) for _ in range(GRP)],
            pltpu.VMEM((GRP, 128, c), jnp.float32),
            pltpu.SemaphoreType.DMA,
            pltpu.VMEM_SHARED((n_nodes, c), jnp.float32),
        ],
    )
    def sc_kern(y_hbm, src_hbm, dst_hbm, kidx_hbm, zeros_hbm, out_hbm,
                srcv, kidxv, flats, dsts, rows, sem, acc):
        cid = lax.axis_index("c")
        sid = lax.axis_index("s")
        w = sid * ncores + cid

        r0 = sid * rpt
        pltpu.sync_copy(zeros_hbm.at[pl.ds(r0, rpt)], acc.at[pl.ds(r0, rpt)])
        if tail:
            @pl.when(sid == 0)
            def _tail_init():
                pltpu.sync_copy(zeros_hbm.at[pl.ds(nsub * rpt, tail)],
                                acc.at[pl.ds(nsub * rpt, tail)])
        plsc.subcore_barrier()

        def super_body(j, carry):
            sc_idx = w + nw * j

            @pl.when(sc_idx < nsuper)
            def _():
                base = sc_idx * SUP
                pltpu.sync_copy(src_hbm.at[pl.ds(base, SUP)], srcv)
                pltpu.sync_copy(kidx_hbm.at[pl.ds(base, SUP)], kidxv)
                for g in range(GRP):
                    pltpu.sync_copy(dst_hbm.at[pl.ds(base + g * 128, 128)],
                                    dsts[g])
                    for l in range(8):
                        sl = pl.ds(g * 128 + l * 16, 16)
                        flats[g][pl.ds(l * 16, 16)] = (
                            kidxv[sl] * n_nodes + srcv[sl])
                cps = [pltpu.async_copy(y_hbm.at[flats[g]], rows.at[g], sem)
                       for g in range(GRP)]
                for g in range(GRP):
                    cps[g].wait()
                for g in range(GRP):
                    pltpu.sync_copy(rows.at[g], acc.at[dsts[g]], add=True)

            return carry

        lax.fori_loop(0, iters, super_body, 0)
        plsc.subcore_barrier()
        pltpu.sync_copy(acc.at[pl.ds(r0, rpt)],
                        out_hbm.at[cid, pl.ds(r0, rpt)])
        if tail:
            @pl.when(sid == 0)
            def _tail_out():
                pltpu.sync_copy(acc.at[pl.ds(nsub * rpt, tail)],
                                out_hbm.at[cid, pl.ds(nsub * rpt, tail)])

    return sc_kern


def kernel(x, edge_index, kernel_idx, W1, gamma1, beta1, W2, gamma2, beta2):
    n, c = x.shape
    k = W1.shape[0]
    e = kernel_idx.shape[0]
    bn = 2000

    src = edge_index[0].astype(jnp.int32)
    dst = edge_index[1].astype(jnp.int32)
    kidx = kernel_idx.astype(jnp.int32)
    zeros = jnp.zeros((n, c), jnp.float32)

    sc_gs = _make_sc_gather_scatter(k * n, n, c, e)

    y1 = _einsum_xw(x, W1, bn).reshape(k * n, c)
    p1 = sc_gs(y1, src, dst, kidx, zeros)
    h1, coef1 = _stats(p1, gamma1, beta1, bn)
    y2 = _einsum_bn_xw(h1, coef1, W2, bn).reshape(k * n, c)
    p2 = sc_gs(y2, src, dst, kidx, zeros)
    h2, coef2 = _stats(p2, gamma2, beta2, bn)
    return _final(h2, x, coef2, bn)


# speedup vs baseline: 4.1829x; 1.4943x over previous
import functools

import jax
import jax.numpy as jnp
from jax import lax
from jax.experimental import pallas as pl
from jax.experimental.pallas import tpu as pltpu
from jax.experimental.pallas import tpu_sc as plsc

_EPS = 1e-5


def _xw_body(x_ref, w_ref, o_ref):
    o_ref[0] = jnp.dot(x_ref[...], w_ref[0], preferred_element_type=jnp.float32)


def _einsum_xw(x, w, bn):
    n, c = x.shape
    k = w.shape[0]
    nb = n // bn
    return pl.pallas_call(
        _xw_body,
        grid=(nb, k),
        in_specs=[
            pl.BlockSpec((bn, c), lambda i, j: (i, 0)),
            pl.BlockSpec((1, c, c), lambda i, j: (j, 0, 0)),
        ],
        out_specs=pl.BlockSpec((1, bn, c), lambda i, j: (j, i, 0)),
        out_shape=jax.ShapeDtypeStruct((k, n, c), jnp.float32),
    )(x, w)


def _bnxw_body(h_ref, coef_ref, w_ref, o_ref):
    c1 = coef_ref[0:1, :]
    c2 = coef_ref[1:2, :]
    a = jnp.maximum(h_ref[...] * c1 + c2, 0.0)
    o_ref[0] = jnp.dot(a, w_ref[0], preferred_element_type=jnp.float32)


def _einsum_bn_xw(h, coef, w, bn):
    n, c = h.shape
    k = w.shape[0]
    nb = n // bn
    return pl.pallas_call(
        _bnxw_body,
        grid=(nb, k),
        in_specs=[
            pl.BlockSpec((bn, c), lambda i, j: (i, 0)),
            pl.BlockSpec((8, c), lambda i, j: (0, 0)),
            pl.BlockSpec((1, c, c), lambda i, j: (j, 0, 0)),
        ],
        out_specs=pl.BlockSpec((1, bn, c), lambda i, j: (j, i, 0)),
        out_shape=jax.ShapeDtypeStruct((k, n, c), jnp.float32),
    )(h, coef, w)


def _stats_body(n_nodes, p_ref, g_ref, b_ref, h_ref, coef_ref):
    h = p_ref[0] + p_ref[1]
    h_ref[...] = h

    @pl.when(pl.program_id(0) == 0)
    def _init():
        coef_ref[...] = jnp.zeros_like(coef_ref)

    coef_ref[0:1, :] = coef_ref[0:1, :] + jnp.sum(h, axis=0, keepdims=True)
    coef_ref[1:2, :] = coef_ref[1:2, :] + jnp.sum(h * h, axis=0, keepdims=True)

    @pl.when(pl.program_id(0) == pl.num_programs(0) - 1)
    def _finalize():
        tot = coef_ref[0:1, :]
        totsq = coef_ref[1:2, :]
        mean = tot / n_nodes
        var = totsq / n_nodes - mean * mean
        c1 = g_ref[...] * lax.rsqrt(var + _EPS)
        c2 = b_ref[...] - mean * c1
        coef_ref[0:1, :] = c1
        coef_ref[1:2, :] = c2


def _stats(p, gamma, beta, bn):
    _, n, c = p.shape
    nb = n // bn
    return pl.pallas_call(
        functools.partial(_stats_body, float(n)),
        grid=(nb,),
        in_specs=[
            pl.BlockSpec((2, bn, c), lambda i: (0, i, 0)),
            pl.BlockSpec((1, c), lambda i: (0, 0)),
            pl.BlockSpec((1, c), lambda i: (0, 0)),
        ],
        out_specs=[
            pl.BlockSpec((bn, c), lambda i: (i, 0)),
            pl.BlockSpec((8, c), lambda i: (0, 0)),
        ],
        out_shape=[
            jax.ShapeDtypeStruct((n, c), jnp.float32),
            jax.ShapeDtypeStruct((8, c), jnp.float32),
        ],
    )(p, gamma.reshape(1, c), beta.reshape(1, c))


def _final_body(h_ref, x_ref, coef_ref, o_ref):
    c1 = coef_ref[0:1, :]
    c2 = coef_ref[1:2, :]
    o_ref[...] = jnp.maximum(h_ref[...] * c1 + c2 + x_ref[...], 0.0)


def _final(h, x, coef, bn):
    n, c = h.shape
    nb = n // bn
    return pl.pallas_call(
        _final_body,
        grid=(nb,),
        in_specs=[
            pl.BlockSpec((bn, c), lambda i: (i, 0)),
            pl.BlockSpec((bn, c), lambda i: (i, 0)),
            pl.BlockSpec((8, c), lambda i: (0, 0)),
        ],
        out_specs=pl.BlockSpec((bn, c), lambda i: (i, 0)),
        out_shape=jax.ShapeDtypeStruct((n, c), jnp.float32),
    )(h, x, coef)


def _make_sc_gather_scatter(kn, n_nodes, c, n_edges):
    B = 128
    nchunks = n_edges // B
    info = plsc.get_sparse_core_info()
    ncores, nsub = info.num_cores, info.num_subcores
    nw = ncores * nsub
    iters = (nchunks + nw - 1) // nw
    npair = (iters + 2) // 2
    rpt = (n_nodes // nsub) // 8 * 8
    tail = n_nodes - nsub * rpt

    mesh = plsc.VectorSubcoreMesh(core_axis_name="c", subcore_axis_name="s")
    ivmem = lambda: [pltpu.VMEM((B,), jnp.int32) for _ in range(2)]

    @functools.partial(
        pl.kernel,
        out_type=jax.ShapeDtypeStruct((ncores, n_nodes, c), jnp.float32),
        mesh=mesh,
        scratch_types=[
            ivmem(), ivmem(), ivmem(), ivmem(),
            pltpu.VMEM((2, B, c), jnp.float32),
            [pltpu.SemaphoreType.DMA for _ in range(2)],
            [pltpu.SemaphoreType.DMA for _ in range(2)],
            [pltpu.SemaphoreType.DMA for _ in range(2)],
            pltpu.VMEM_SHARED((n_nodes, c), jnp.float32),
        ],
    )
    def sc_kern(y_hbm, src_hbm, dst_hbm, kidx_hbm, zeros_hbm, out_hbm,
                srcv, kidxv, dstv, flatv, rows, isems, dsems, gsems, acc):
        cid = lax.axis_index("c")
        sid = lax.axis_index("s")
        w = sid * ncores + cid

        r0 = sid * rpt
        pltpu.sync_copy(zeros_hbm.at[pl.ds(r0, rpt)], acc.at[pl.ds(r0, rpt)])
        if tail:
            @pl.when(sid == 0)
            def _tail_init():
                pltpu.sync_copy(zeros_hbm.at[pl.ds(nsub * rpt, tail)],
                                acc.at[pl.ds(nsub * rpt, tail)])
        plsc.subcore_barrier()

        def fire_sk(cj, b):
            base = cj * B
            pltpu.async_copy(src_hbm.at[pl.ds(base, B)], srcv[b], isems[b])
            pltpu.async_copy(kidx_hbm.at[pl.ds(base, B)], kidxv[b], isems[b])

        def fire_dst(cj, b):
            pltpu.async_copy(dst_hbm.at[pl.ds(cj * B, B)], dstv[b], dsems[b])

        def wait_sk(b):
            pltpu.make_async_copy(src_hbm.at[pl.ds(0, B)], srcv[b],
                                  isems[b]).wait()
            pltpu.make_async_copy(kidx_hbm.at[pl.ds(0, B)], kidxv[b],
                                  isems[b]).wait()

        def wait_dst(b):
            pltpu.make_async_copy(dst_hbm.at[pl.ds(0, B)], dstv[b],
                                  dsems[b]).wait()

        def wait_gather(b):
            pltpu.make_async_copy(y_hbm.at[flatv[b]], rows.at[b],
                                  gsems[b]).wait()

        fire_sk(w, 0)
        fire_dst(w, 0)

        def pair_body(jj, carry):
            for b in range(2):
                j = 2 * jj + b
                cj = w + nw * j

                @pl.when(cj < nchunks)
                def _launch():
                    wait_sk(b)
                    for l in range(8):
                        sl = pl.ds(l * 16, 16)
                        flatv[b][sl] = kidxv[b][sl] * n_nodes + srcv[b][sl]
                    pltpu.async_copy(y_hbm.at[flatv[b]], rows.at[b], gsems[b])

                    @pl.when(cj + nw < nchunks)
                    def _prefetch_sk():
                        fire_sk(cj + nw, 1 - b)

                @pl.when((j >= 1) & (cj - nw < nchunks))
                def _consume():
                    wait_gather(1 - b)
                    wait_dst(1 - b)
                    pltpu.sync_copy(rows.at[1 - b], acc.at[dstv[1 - b]],
                                    add=True)

                @pl.when(cj + nw < nchunks)
                def _prefetch_dst():
                    fire_dst(cj + nw, 1 - b)

            return carry

        lax.fori_loop(0, npair, pair_body, 0)
        plsc.subcore_barrier()
        pltpu.sync_copy(acc.at[pl.ds(r0, rpt)],
                        out_hbm.at[cid, pl.ds(r0, rpt)])
        if tail:
            @pl.when(sid == 0)
            def _tail_out():
                pltpu.sync_copy(acc.at[pl.ds(nsub * rpt, tail)],
                                out_hbm.at[cid, pl.ds(nsub * rpt, tail)])

    return sc_kern


def kernel(x, edge_index, kernel_idx, W1, gamma1, beta1, W2, gamma2, beta2):
    n, c = x.shape
    k = W1.shape[0]
    e = kernel_idx.shape[0]
    bn = 2000

    src = edge_index[0].astype(jnp.int32)
    dst = edge_index[1].astype(jnp.int32)
    kidx = kernel_idx.astype(jnp.int32)
    zeros = jnp.zeros((n, c), jnp.float32)

    sc_gs = _make_sc_gather_scatter(k * n, n, c, e)

    y1 = _einsum_xw(x, W1, bn).reshape(k * n, c)
    p1 = sc_gs(y1, src, dst, kidx, zeros)
    h1, coef1 = _stats(p1, gamma1, beta1, bn)
    y2 = _einsum_bn_xw(h1, coef1, W2, bn).reshape(k * n, c)
    p2 = sc_gs(y2, src, dst, kidx, zeros)
    h2, coef2 = _stats(p2, gamma2, beta2, bn)
    return _final(h2, x, coef2, bn)


# speedup vs baseline: 5.9121x; 1.4134x over previous
import functools

import jax
import jax.numpy as jnp
from jax import lax
from jax.experimental import pallas as pl
from jax.experimental.pallas import tpu as pltpu
from jax.experimental.pallas import tpu_sc as plsc

_EPS = 1e-5


def _xw_body(x_ref, w_ref, o_ref):
    o_ref[0] = jnp.dot(x_ref[...], w_ref[0], preferred_element_type=jnp.float32)


def _einsum_xw(x, w):
    n, c = x.shape
    k = w.shape[0]
    return pl.pallas_call(
        _xw_body,
        grid=(k,),
        in_specs=[
            pl.BlockSpec((n, c), lambda j: (0, 0)),
            pl.BlockSpec((1, c, c), lambda j: (j, 0, 0)),
        ],
        out_specs=pl.BlockSpec((1, n, c), lambda j: (j, 0, 0)),
        out_shape=jax.ShapeDtypeStruct((k, n, c), jnp.float32),
    )(x, w)


def _bnxw_body(h_ref, coef_ref, w_ref, o_ref):
    c1 = coef_ref[0:1, :]
    c2 = coef_ref[1:2, :]
    a = jnp.maximum(h_ref[...] * c1 + c2, 0.0)
    o_ref[0] = jnp.dot(a, w_ref[0], preferred_element_type=jnp.float32)


def _einsum_bn_xw(h, coef, w):
    n, c = h.shape
    k = w.shape[0]
    return pl.pallas_call(
        _bnxw_body,
        grid=(k,),
        in_specs=[
            pl.BlockSpec((n, c), lambda j: (0, 0)),
            pl.BlockSpec((8, c), lambda j: (0, 0)),
            pl.BlockSpec((1, c, c), lambda j: (j, 0, 0)),
        ],
        out_specs=pl.BlockSpec((1, n, c), lambda j: (j, 0, 0)),
        out_shape=jax.ShapeDtypeStruct((k, n, c), jnp.float32),
    )(h, coef, w)


def _stats_body(n_nodes, p_ref, g_ref, b_ref, h_ref, coef_ref):
    h = p_ref[0] + p_ref[1]
    h_ref[...] = h

    @pl.when(pl.program_id(0) == 0)
    def _init():
        coef_ref[...] = jnp.zeros_like(coef_ref)

    coef_ref[0:1, :] = coef_ref[0:1, :] + jnp.sum(h, axis=0, keepdims=True)
    coef_ref[1:2, :] = coef_ref[1:2, :] + jnp.sum(h * h, axis=0, keepdims=True)

    @pl.when(pl.program_id(0) == pl.num_programs(0) - 1)
    def _finalize():
        tot = coef_ref[0:1, :]
        totsq = coef_ref[1:2, :]
        mean = tot / n_nodes
        var = totsq / n_nodes - mean * mean
        c1 = g_ref[...] * lax.rsqrt(var + _EPS)
        c2 = b_ref[...] - mean * c1
        coef_ref[0:1, :] = c1
        coef_ref[1:2, :] = c2


def _stats(p, gamma, beta, bn):
    _, n, c = p.shape
    nb = n // bn
    return pl.pallas_call(
        functools.partial(_stats_body, float(n)),
        grid=(nb,),
        in_specs=[
            pl.BlockSpec((2, bn, c), lambda i: (0, i, 0)),
            pl.BlockSpec((1, c), lambda i: (0, 0)),
            pl.BlockSpec((1, c), lambda i: (0, 0)),
        ],
        out_specs=[
            pl.BlockSpec((bn, c), lambda i: (i, 0)),
            pl.BlockSpec((8, c), lambda i: (0, 0)),
        ],
        out_shape=[
            jax.ShapeDtypeStruct((n, c), jnp.float32),
            jax.ShapeDtypeStruct((8, c), jnp.float32),
        ],
    )(p, gamma.reshape(1, c), beta.reshape(1, c))


def _final_body(h_ref, x_ref, coef_ref, o_ref):
    c1 = coef_ref[0:1, :]
    c2 = coef_ref[1:2, :]
    o_ref[...] = jnp.maximum(h_ref[...] * c1 + c2 + x_ref[...], 0.0)


def _final(h, x, coef, bn):
    n, c = h.shape
    nb = n // bn
    return pl.pallas_call(
        _final_body,
        grid=(nb,),
        in_specs=[
            pl.BlockSpec((bn, c), lambda i: (i, 0)),
            pl.BlockSpec((bn, c), lambda i: (i, 0)),
            pl.BlockSpec((8, c), lambda i: (0, 0)),
        ],
        out_specs=pl.BlockSpec((bn, c), lambda i: (i, 0)),
        out_shape=jax.ShapeDtypeStruct((n, c), jnp.float32),
    )(h, x, coef)


def _make_sc_gather_scatter(kn, n_nodes, c, n_edges):
    B = 128
    nchunks = n_edges // B
    info = plsc.get_sparse_core_info()
    ncores, nsub = info.num_cores, info.num_subcores
    nw = ncores * nsub
    iters = (nchunks + nw - 1) // nw
    npair = (iters + 2) // 2
    rpt = (n_nodes // nsub) // 8 * 8
    tail = n_nodes - nsub * rpt

    mesh = plsc.VectorSubcoreMesh(core_axis_name="c", subcore_axis_name="s")
    ivmem = lambda: [pltpu.VMEM((B,), jnp.int32) for _ in range(2)]

    @functools.partial(
        pl.kernel,
        out_type=jax.ShapeDtypeStruct((ncores, n_nodes, c), jnp.float32),
        mesh=mesh,
        scratch_types=[
            ivmem(), ivmem(), ivmem(), ivmem(),
            pltpu.VMEM((2, B, c), jnp.float32),
            [pltpu.SemaphoreType.DMA for _ in range(2)],
            [pltpu.SemaphoreType.DMA for _ in range(2)],
            [pltpu.SemaphoreType.DMA for _ in range(2)],
            pltpu.VMEM_SHARED((n_nodes, c), jnp.float32),
        ],
    )
    def sc_kern(y_hbm, ei_hbm, kidx_hbm, zeros_hbm, out_hbm,
                srcv, kidxv, dstv, flatv, rows, isems, dsems, gsems, acc):
        cid = lax.axis_index("c")
        sid = lax.axis_index("s")
        w = sid * ncores + cid

        r0 = sid * rpt
        pltpu.sync_copy(zeros_hbm.at[pl.ds(r0, rpt)], acc.at[pl.ds(r0, rpt)])
        if tail:
            @pl.when(sid == 0)
            def _tail_init():
                pltpu.sync_copy(zeros_hbm.at[pl.ds(nsub * rpt, tail)],
                                acc.at[pl.ds(nsub * rpt, tail)])
        plsc.subcore_barrier()

        def fire_sk(cj, b):
            base = cj * B
            pltpu.async_copy(ei_hbm.at[0, pl.ds(base, B)], srcv[b], isems[b])
            pltpu.async_copy(kidx_hbm.at[pl.ds(base, B)], kidxv[b], isems[b])

        def fire_dst(cj, b):
            pltpu.async_copy(ei_hbm.at[1, pl.ds(cj * B, B)], dstv[b], dsems[b])

        def wait_sk(b):
            pltpu.make_async_copy(ei_hbm.at[0, pl.ds(0, B)], srcv[b],
                                  isems[b]).wait()
            pltpu.make_async_copy(kidx_hbm.at[pl.ds(0, B)], kidxv[b],
                                  isems[b]).wait()

        def wait_dst(b):
            pltpu.make_async_copy(ei_hbm.at[1, pl.ds(0, B)], dstv[b],
                                  dsems[b]).wait()

        def wait_gather(b):
            pltpu.make_async_copy(y_hbm.at[flatv[b]], rows.at[b],
                                  gsems[b]).wait()

        fire_sk(w, 0)
        fire_dst(w, 0)

        def pair_body(jj, carry):
            for b in range(2):
                j = 2 * jj + b
                cj = w + nw * j

                @pl.when(cj < nchunks)
                def _launch():
                    wait_sk(b)
                    for l in range(8):
                        sl = pl.ds(l * 16, 16)
                        flatv[b][sl] = kidxv[b][sl] * n_nodes + srcv[b][sl]
                    pltpu.async_copy(y_hbm.at[flatv[b]], rows.at[b], gsems[b])

                    @pl.when(cj + nw < nchunks)
                    def _prefetch_sk():
                        fire_sk(cj + nw, 1 - b)

                @pl.when((j >= 1) & (cj - nw < nchunks))
                def _consume():
                    wait_gather(1 - b)
                    wait_dst(1 - b)
                    pltpu.sync_copy(rows.at[1 - b], acc.at[dstv[1 - b]],
                                    add=True)

                @pl.when(cj + nw < nchunks)
                def _prefetch_dst():
                    fire_dst(cj + nw, 1 - b)

            return carry

        lax.fori_loop(0, npair, pair_body, 0)
        plsc.subcore_barrier()
        pltpu.sync_copy(acc.at[pl.ds(r0, rpt)],
                        out_hbm.at[cid, pl.ds(r0, rpt)])
        if tail:
            @pl.when(sid == 0)
            def _tail_out():
                pltpu.sync_copy(acc.at[pl.ds(nsub * rpt, tail)],
                                out_hbm.at[cid, pl.ds(nsub * rpt, tail)])

    return sc_kern


def kernel(x, edge_index, kernel_idx, W1, gamma1, beta1, W2, gamma2, beta2):
    n, c = x.shape
    k = W1.shape[0]
    e = kernel_idx.shape[0]
    bn = 2000

    ei = edge_index.astype(jnp.int32)
    kidx = kernel_idx.astype(jnp.int32)
    zeros = jnp.zeros((n, c), jnp.float32)

    sc_gs = _make_sc_gather_scatter(k * n, n, c, e)

    y1 = _einsum_xw(x, W1).reshape(k * n, c)
    p1 = sc_gs(y1, ei, kidx, zeros)
    h1, coef1 = _stats(p1, gamma1, beta1, bn)
    y2 = _einsum_bn_xw(h1, coef1, W2).reshape(k * n, c)
    p2 = sc_gs(y2, ei, kidx, zeros)
    h2, coef2 = _stats(p2, gamma2, beta2, bn)
    return _final(h2, x, coef2, bn)


# speedup vs baseline: 6.2708x; 1.0607x over previous
import functools

import jax
import jax.numpy as jnp
from jax import lax
from jax.experimental import pallas as pl
from jax.experimental.pallas import tpu as pltpu
from jax.experimental.pallas import tpu_sc as plsc

_EPS = 1e-5


def _xw_body(x_ref, w_ref, o_ref):
    o_ref[0] = jnp.dot(x_ref[...], w_ref[0], preferred_element_type=jnp.float32)


def _einsum_xw(x, w):
    n, c = x.shape
    k = w.shape[0]
    return pl.pallas_call(
        _xw_body,
        grid=(k,),
        in_specs=[
            pl.BlockSpec((n, c), lambda j: (0, 0)),
            pl.BlockSpec((1, c, c), lambda j: (j, 0, 0)),
        ],
        out_specs=pl.BlockSpec((1, n, c), lambda j: (j, 0, 0)),
        out_shape=jax.ShapeDtypeStruct((k, n, c), jnp.float32),
    )(x, w)


def _bnxw_body(h_ref, coef_ref, w_ref, o_ref):
    c1 = coef_ref[0:1, :]
    c2 = coef_ref[1:2, :]
    a = jnp.maximum(h_ref[...] * c1 + c2, 0.0)
    o_ref[0] = jnp.dot(a, w_ref[0], preferred_element_type=jnp.float32)


def _einsum_bn_xw(h, coef, w):
    n, c = h.shape
    k = w.shape[0]
    return pl.pallas_call(
        _bnxw_body,
        grid=(k,),
        in_specs=[
            pl.BlockSpec((n, c), lambda j: (0, 0)),
            pl.BlockSpec((8, c), lambda j: (0, 0)),
            pl.BlockSpec((1, c, c), lambda j: (j, 0, 0)),
        ],
        out_specs=pl.BlockSpec((1, n, c), lambda j: (j, 0, 0)),
        out_shape=jax.ShapeDtypeStruct((k, n, c), jnp.float32),
    )(h, coef, w)


def _stats_body(n_nodes, p_ref, g_ref, b_ref, h_ref, coef_ref):
    h = p_ref[0] + p_ref[1]
    h_ref[...] = h

    @pl.when(pl.program_id(0) == 0)
    def _init():
        coef_ref[...] = jnp.zeros_like(coef_ref)

    coef_ref[0:1, :] = coef_ref[0:1, :] + jnp.sum(h, axis=0, keepdims=True)
    coef_ref[1:2, :] = coef_ref[1:2, :] + jnp.sum(h * h, axis=0, keepdims=True)

    @pl.when(pl.program_id(0) == pl.num_programs(0) - 1)
    def _finalize():
        tot = coef_ref[0:1, :]
        totsq = coef_ref[1:2, :]
        mean = tot / n_nodes
        var = totsq / n_nodes - mean * mean
        c1 = g_ref[...] * lax.rsqrt(var + _EPS)
        c2 = b_ref[...] - mean * c1
        coef_ref[0:1, :] = c1
        coef_ref[1:2, :] = c2


def _stats(p, gamma, beta, bn):
    _, n, c = p.shape
    nb = n // bn
    return pl.pallas_call(
        functools.partial(_stats_body, float(n)),
        grid=(nb,),
        in_specs=[
            pl.BlockSpec((2, bn, c), lambda i: (0, i, 0)),
            pl.BlockSpec((1, c), lambda i: (0, 0)),
            pl.BlockSpec((1, c), lambda i: (0, 0)),
        ],
        out_specs=[
            pl.BlockSpec((bn, c), lambda i: (i, 0)),
            pl.BlockSpec((8, c), lambda i: (0, 0)),
        ],
        out_shape=[
            jax.ShapeDtypeStruct((n, c), jnp.float32),
            jax.ShapeDtypeStruct((8, c), jnp.float32),
        ],
    )(p, gamma.reshape(1, c), beta.reshape(1, c))


def _final_body(h_ref, x_ref, coef_ref, o_ref):
    c1 = coef_ref[0:1, :]
    c2 = coef_ref[1:2, :]
    o_ref[...] = jnp.maximum(h_ref[...] * c1 + c2 + x_ref[...], 0.0)


def _final(h, x, coef, bn):
    n, c = h.shape
    nb = n // bn
    return pl.pallas_call(
        _final_body,
        grid=(nb,),
        in_specs=[
            pl.BlockSpec((bn, c), lambda i: (i, 0)),
            pl.BlockSpec((bn, c), lambda i: (i, 0)),
            pl.BlockSpec((8, c), lambda i: (0, 0)),
        ],
        out_specs=pl.BlockSpec((bn, c), lambda i: (i, 0)),
        out_shape=jax.ShapeDtypeStruct((n, c), jnp.float32),
    )(h, x, coef)


def _make_sc_gather_scatter(kn, n_nodes, c, n_edges):
    B = 128
    U = 6
    nchunks = n_edges // B
    info = plsc.get_sparse_core_info()
    ncores, nsub = info.num_cores, info.num_subcores
    nw = ncores * nsub
    iters = (nchunks + nw - 1) // nw
    nloop = (iters + 2 + U - 1) // U + 1
    rpt = (n_nodes // nsub) // 8 * 8
    tail = n_nodes - nsub * rpt

    mesh = plsc.VectorSubcoreMesh(core_axis_name="c", subcore_axis_name="s")
    ivmem = lambda r: [pltpu.VMEM((B,), jnp.int32) for _ in range(r)]
    sems = lambda r: [pltpu.SemaphoreType.DMA for _ in range(r)]

    @functools.partial(
        pl.kernel,
        out_type=jax.ShapeDtypeStruct((ncores, n_nodes, c), jnp.float32),
        mesh=mesh,
        scratch_types=[
            ivmem(2), ivmem(2), ivmem(6), ivmem(2),
            pltpu.VMEM((3, B, c), jnp.float32),
            sems(2), sems(6), sems(3), sems(3),
            pltpu.VMEM_SHARED((n_nodes, c), jnp.float32),
        ],
    )
    def sc_kern(y_hbm, ei_hbm, kidx_hbm, out_hbm,
                srcv, kidxv, dstv, flatv, rows, isems, dsems, gsems, ssems,
                acc):
        cid = lax.axis_index("c")
        sid = lax.axis_index("s")
        w = sid * ncores + cid

        zsrc = rows.at[0]
        for r in range(B):
            for l in range(c // 16):
                zsrc[r, pl.ds(l * 16, 16)] = jnp.zeros((16,), jnp.float32)
        r0 = sid * rpt
        done = 0
        while done < rpt:
            step = min(B, rpt - done)
            pltpu.sync_copy(zsrc.at[pl.ds(0, step)],
                            acc.at[pl.ds(r0 + done, step)])
            done += step
        if tail:
            @pl.when(sid == 0)
            def _tail_init():
                pltpu.sync_copy(zsrc.at[pl.ds(0, tail)],
                                acc.at[pl.ds(nsub * rpt, tail)])
        plsc.subcore_barrier()

        def fire_sk(cj, b):
            base = cj * B
            pltpu.async_copy(ei_hbm.at[0, pl.ds(base, B)], srcv[b], isems[b])
            pltpu.async_copy(kidx_hbm.at[pl.ds(base, B)], kidxv[b], isems[b])

        def fire_dst(cj, d):
            pltpu.async_copy(ei_hbm.at[1, pl.ds(cj * B, B)], dstv[d], dsems[d])

        def wait_sk(b):
            pltpu.make_async_copy(ei_hbm.at[0, pl.ds(0, B)], srcv[b],
                                  isems[b]).wait()
            pltpu.make_async_copy(kidx_hbm.at[pl.ds(0, B)], kidxv[b],
                                  isems[b]).wait()

        def wait_dst(d):
            pltpu.make_async_copy(ei_hbm.at[1, pl.ds(0, B)], dstv[d],
                                  dsems[d]).wait()

        def wait_gather(g):
            pltpu.make_async_copy(y_hbm.at[flatv[g % 2]], rows.at[g],
                                  gsems[g]).wait()

        def wait_scatter(g, d):
            pltpu.make_async_copy(rows.at[g], acc.at[dstv[d]],
                                  ssems[g]).wait()

        fire_sk(w, 0)
        fire_dst(w, 0)

        def loop_body(jj, carry):
            for u in range(U):
                j = U * jj + u
                cj = w + nw * j
                b = u % 2
                g = u % 3
                d = u % 6

                @pl.when((j >= 2) & (cj - 2 * nw < nchunks))
                def _consume():
                    wait_gather((u - 2) % 3)
                    wait_dst((u - 2) % 6)
                    pltpu.async_copy(rows.at[(u - 2) % 3],
                                     acc.at[dstv[(u - 2) % 6]],
                                     ssems[(u - 2) % 3], add=True)

                @pl.when(cj < nchunks)
                def _launch():
                    wait_sk(b)
                    for l in range(8):
                        sl = pl.ds(l * 16, 16)
                        flatv[b][sl] = kidxv[b][sl] * n_nodes + srcv[b][sl]

                    @pl.when(j >= 3)
                    def _free_rows():
                        wait_scatter(g, (u - 3) % 6)

                    pltpu.async_copy(y_hbm.at[flatv[b]], rows.at[g], gsems[g])

                @pl.when(cj + nw < nchunks)
                def _prefetch():
                    fire_sk(cj + nw, 1 - b)
                    fire_dst(cj + nw, (u + 1) % 6)

            return carry

        lax.fori_loop(0, nloop, loop_body, 0)
        plsc.subcore_barrier()
        pltpu.sync_copy(acc.at[pl.ds(r0, rpt)],
                        out_hbm.at[cid, pl.ds(r0, rpt)])
        if tail:
            @pl.when(sid == 0)
            def _tail_out():
                pltpu.sync_copy(acc.at[pl.ds(nsub * rpt, tail)],
                                out_hbm.at[cid, pl.ds(nsub * rpt, tail)])

    return sc_kern


def kernel(x, edge_index, kernel_idx, W1, gamma1, beta1, W2, gamma2, beta2):
    n, c = x.shape
    k = W1.shape[0]
    e = kernel_idx.shape[0]
    bn = 2000

    ei = edge_index.astype(jnp.int32)
    kidx = kernel_idx.astype(jnp.int32)

    sc_gs = _make_sc_gather_scatter(k * n, n, c, e)

    y1 = _einsum_xw(x, W1).reshape(k * n, c)
    p1 = sc_gs(y1, ei, kidx)
    h1, coef1 = _stats(p1, gamma1, beta1, bn)
    y2 = _einsum_bn_xw(h1, coef1, W2).reshape(k * n, c)
    p2 = sc_gs(y2, ei, kidx)
    h2, coef2 = _stats(p2, gamma2, beta2, bn)
    return _final(h2, x, coef2, bn)


# speedup vs baseline: 6.3104x; 1.0063x over previous
import functools

import jax
import jax.numpy as jnp
from jax import lax
from jax.experimental import pallas as pl
from jax.experimental.pallas import tpu as pltpu
from jax.experimental.pallas import tpu_sc as plsc

_EPS = 1e-5


def _xw_body(x_ref, w_ref, o_ref):
    o_ref[0] = jnp.dot(x_ref[...], w_ref[0], preferred_element_type=jnp.float32)


def _einsum_xw(x, w):
    n, c = x.shape
    k = w.shape[0]
    return pl.pallas_call(
        _xw_body,
        grid=(k,),
        in_specs=[
            pl.BlockSpec((n, c), lambda j: (0, 0)),
            pl.BlockSpec((1, c, c), lambda j: (j, 0, 0)),
        ],
        out_specs=pl.BlockSpec((1, n, c), lambda j: (j, 0, 0)),
        out_shape=jax.ShapeDtypeStruct((k, n, c), jnp.float32),
    )(x, w)


def _bnxw_body(h_ref, coef_ref, w_ref, o_ref):
    c1 = coef_ref[0:1, :]
    c2 = coef_ref[1:2, :]
    a = jnp.maximum(h_ref[...] * c1 + c2, 0.0)
    o_ref[0] = jnp.dot(a, w_ref[0], preferred_element_type=jnp.float32)


def _einsum_bn_xw(h, coef, w):
    n, c = h.shape
    k = w.shape[0]
    return pl.pallas_call(
        _bnxw_body,
        grid=(k,),
        in_specs=[
            pl.BlockSpec((n, c), lambda j: (0, 0)),
            pl.BlockSpec((8, c), lambda j: (0, 0)),
            pl.BlockSpec((1, c, c), lambda j: (j, 0, 0)),
        ],
        out_specs=pl.BlockSpec((1, n, c), lambda j: (j, 0, 0)),
        out_shape=jax.ShapeDtypeStruct((k, n, c), jnp.float32),
    )(h, coef, w)


def _stats_body(n_nodes, p_ref, g_ref, b_ref, h_ref, coef_ref):
    h = p_ref[0] + p_ref[1]
    h_ref[...] = h

    @pl.when(pl.program_id(0) == 0)
    def _init():
        coef_ref[...] = jnp.zeros_like(coef_ref)

    coef_ref[0:1, :] = coef_ref[0:1, :] + jnp.sum(h, axis=0, keepdims=True)
    coef_ref[1:2, :] = coef_ref[1:2, :] + jnp.sum(h * h, axis=0, keepdims=True)

    @pl.when(pl.program_id(0) == pl.num_programs(0) - 1)
    def _finalize():
        tot = coef_ref[0:1, :]
        totsq = coef_ref[1:2, :]
        mean = tot / n_nodes
        var = totsq / n_nodes - mean * mean
        c1 = g_ref[...] * lax.rsqrt(var + _EPS)
        c2 = b_ref[...] - mean * c1
        coef_ref[0:1, :] = c1
        coef_ref[1:2, :] = c2


def _stats(p, gamma, beta, bn):
    _, n, c = p.shape
    nb = n // bn
    return pl.pallas_call(
        functools.partial(_stats_body, float(n)),
        grid=(nb,),
        in_specs=[
            pl.BlockSpec((2, bn, c), lambda i: (0, i, 0)),
            pl.BlockSpec((1, c), lambda i: (0, 0)),
            pl.BlockSpec((1, c), lambda i: (0, 0)),
        ],
        out_specs=[
            pl.BlockSpec((bn, c), lambda i: (i, 0)),
            pl.BlockSpec((8, c), lambda i: (0, 0)),
        ],
        out_shape=[
            jax.ShapeDtypeStruct((n, c), jnp.float32),
            jax.ShapeDtypeStruct((8, c), jnp.float32),
        ],
    )(p, gamma.reshape(1, c), beta.reshape(1, c))


def _final_body(h_ref, x_ref, coef_ref, o_ref):
    c1 = coef_ref[0:1, :]
    c2 = coef_ref[1:2, :]
    o_ref[...] = jnp.maximum(h_ref[...] * c1 + c2 + x_ref[...], 0.0)


def _final(h, x, coef, bn):
    n, c = h.shape
    nb = n // bn
    return pl.pallas_call(
        _final_body,
        grid=(nb,),
        in_specs=[
            pl.BlockSpec((bn, c), lambda i: (i, 0)),
            pl.BlockSpec((bn, c), lambda i: (i, 0)),
            pl.BlockSpec((8, c), lambda i: (0, 0)),
        ],
        out_specs=pl.BlockSpec((bn, c), lambda i: (i, 0)),
        out_shape=jax.ShapeDtypeStruct((n, c), jnp.float32),
    )(h, x, coef)


def _make_sc_gather_scatter(kn, n_nodes, c, n_edges):
    B = 128
    U = 6
    nchunks = n_edges // B
    info = plsc.get_sparse_core_info()
    ncores, nsub = info.num_cores, info.num_subcores
    nw = ncores * nsub
    iters = (nchunks + nw - 1) // nw
    nloop = (iters + 2 + U - 1) // U + 1
    rpt = (n_nodes // nsub) // 8 * 8
    tail = n_nodes - nsub * rpt

    mesh = plsc.VectorSubcoreMesh(core_axis_name="c", subcore_axis_name="s")
    ivmem = lambda r: [pltpu.VMEM((B,), jnp.int32) for _ in range(r)]
    sems = lambda r: [pltpu.SemaphoreType.DMA for _ in range(r)]

    @functools.partial(
        pl.kernel,
        out_type=jax.ShapeDtypeStruct((ncores, n_nodes, c), jnp.float32),
        mesh=mesh,
        scratch_types=[
            ivmem(2), ivmem(2), ivmem(6), ivmem(2),
            pltpu.VMEM((3, B, c), jnp.float32),
            sems(2), sems(6), sems(3), sems(3),
            pltpu.VMEM_SHARED((n_nodes, c), jnp.float32),
        ],
    )
    def sc_kern(y_hbm, ei_hbm, kidx_hbm, out_hbm,
                srcv, kidxv, dstv, flatv, rows, isems, dsems, gsems, ssems,
                acc):
        cid = lax.axis_index("c")
        sid = lax.axis_index("s")
        w = sid * ncores + cid

        zsrc = rows.at[0]
        for r in range(B):
            for l in range(c // 16):
                zsrc[r, pl.ds(l * 16, 16)] = jnp.zeros((16,), jnp.float32)
        r0 = sid * rpt
        done = 0
        while done < rpt:
            step = min(B, rpt - done)
            pltpu.sync_copy(zsrc.at[pl.ds(0, step)],
                            acc.at[pl.ds(r0 + done, step)])
            done += step
        if tail:
            @pl.when(sid == 0)
            def _tail_init():
                pltpu.sync_copy(zsrc.at[pl.ds(0, tail)],
                                acc.at[pl.ds(nsub * rpt, tail)])
        plsc.subcore_barrier()

        def fire_sk(cj, b):
            base = cj * B
            pltpu.async_copy(ei_hbm.at[0, pl.ds(base, B)], srcv[b], isems[b])
            pltpu.async_copy(kidx_hbm.at[pl.ds(base, B)], kidxv[b], isems[b])

        def fire_dst(cj, d):
            pltpu.async_copy(ei_hbm.at[1, pl.ds(cj * B, B)], dstv[d], dsems[d])

        def wait_sk(b):
            pltpu.make_async_copy(ei_hbm.at[0, pl.ds(0, B)], srcv[b],
                                  isems[b]).wait()
            pltpu.make_async_copy(kidx_hbm.at[pl.ds(0, B)], kidxv[b],
                                  isems[b]).wait()

        def wait_dst(d):
            pltpu.make_async_copy(ei_hbm.at[1, pl.ds(0, B)], dstv[d],
                                  dsems[d]).wait()

        def wait_gather(g):
            pltpu.make_async_copy(y_hbm.at[flatv[g % 2]], rows.at[g],
                                  gsems[g]).wait()

        def wait_scatter(g, d):
            pltpu.make_async_copy(rows.at[g], acc.at[dstv[d]],
                                  ssems[g]).wait()

        fire_sk(w, 0)
        fire_dst(w, 0)

        def loop_body(jj, carry):
            for u in range(U):
                j = U * jj + u
                cj = w + nw * j
                b = u % 2
                g = u % 3
                d = u % 6

                @pl.when((j >= 2) & (cj - 2 * nw < nchunks))
                def _consume():
                    wait_gather((u - 2) % 3)
                    wait_dst((u - 2) % 6)
                    pltpu.async_copy(rows.at[(u - 2) % 3],
                                     acc.at[dstv[(u - 2) % 6]],
                                     ssems[(u - 2) % 3], add=True)

                @pl.when(cj < nchunks)
                def _launch():
                    wait_sk(b)
                    for l in range(8):
                        sl = pl.ds(l * 16, 16)
                        flatv[b][sl] = kidxv[b][sl] * n_nodes + srcv[b][sl]

                    @pl.when(j >= 3)
                    def _free_rows():
                        wait_scatter(g, (u - 3) % 6)

                    pltpu.async_copy(y_hbm.at[flatv[b]], rows.at[g], gsems[g])

                @pl.when(cj + nw < nchunks)
                def _prefetch():
                    fire_sk(cj + nw, 1 - b)
                    fire_dst(cj + nw, (u + 1) % 6)

            return carry

        lax.fori_loop(0, nloop, loop_body, 0)
        for mm in range(max(0, iters - 4), iters):
            cm = w + nw * mm

            @pl.when((cm < nchunks) & (cm + 3 * nw >= nchunks))
            def _drain():
                wait_scatter(mm % 3, mm % 6)

        plsc.subcore_barrier()
        pltpu.sync_copy(acc.at[pl.ds(r0, rpt)],
                        out_hbm.at[cid, pl.ds(r0, rpt)])
        if tail:
            @pl.when(sid == 0)
            def _tail_out():
                pltpu.sync_copy(acc.at[pl.ds(nsub * rpt, tail)],
                                out_hbm.at[cid, pl.ds(nsub * rpt, tail)])

    return sc_kern


def kernel(x, edge_index, kernel_idx, W1, gamma1, beta1, W2, gamma2, beta2):
    n, c = x.shape
    k = W1.shape[0]
    e = kernel_idx.shape[0]
    bn = 2000

    ei = edge_index.astype(jnp.int32)
    kidx = kernel_idx.astype(jnp.int32)

    sc_gs = _make_sc_gather_scatter(k * n, n, c, e)

    y1 = _einsum_xw(x, W1).reshape(k * n, c)
    p1 = sc_gs(y1, ei, kidx)
    h1, coef1 = _stats(p1, gamma1, beta1, bn)
    y2 = _einsum_bn_xw(h1, coef1, W2).reshape(k * n, c)
    p2 = sc_gs(y2, ei, kidx)
    h2, coef2 = _stats(p2, gamma2, beta2, bn)
    return _final(h2, x, coef2, bn)
